# baseline (speedup 1.0000x reference)
import functools

import numpy as np

import jax
import jax.numpy as jnp
from jax import lax
from jax.experimental import pallas as pl
from jax.experimental.pallas import tpu as pltpu
from jax.experimental.pallas import tpu_sc as plsc

_NC, _NS, _LANES = 2, 16, 16
_NW = _NC * _NS


def _edge_mlp_body(attr_ref, w123t_ref, w4t_ref, ea_ref):
    t = jnp.dot(attr_ref[...], w123t_ref[...], preferred_element_type=jnp.float32)
    h = jax.nn.relu(t[:, :32])
    g = jnp.tanh(t[:, 32:64]) * jnp.tanh(t[:, 64:96])
    tmp = jnp.concatenate([h, g], axis=1)
    ea_ref[...] = jax.nn.relu(
        jnp.dot(tmp, w4t_ref[...], preferred_element_type=jnp.float32))


def _project_body(x_ref, wz_ref, w11t_ref, b11_ref, w12t_ref, b12_ref,
                  z_ref, r_ref):
    x = x_ref[...]
    z_ref[...] = jnp.dot(
        x, wz_ref[...], preferred_element_type=jnp.float32).astype(jnp.bfloat16)
    r_ref[...] = (
        jnp.tanh(jnp.dot(x, w11t_ref[...], preferred_element_type=jnp.float32)
                 + b11_ref[...])
        * jnp.tanh(jnp.dot(x, w12t_ref[...], preferred_element_type=jnp.float32)
                   + b12_ref[...]))


def _combine_body(p0_ref, p1_ref, bias_ref, r_ref, out_ref):
    left = jax.nn.relu(p0_ref[...] + p1_ref[...] + bias_ref[...])
    out_ref[...] = jnp.concatenate([left, r_ref[...]], axis=1)


def kernel(x, edge_index, edge_attr, fc1_1_w, fc1_2_w, fc1_3_w, fc1_4_w,
           conv_weight, conv_bias, fc11_w, fc11_b, fc12_w, fc12_b):
    n, ninp = x.shape
    e = edge_attr.shape[0]
    k_sup, _, nout1 = conv_weight.shape
    nout2 = fc11_w.shape[0]
    d = k_sup * nout1
    nf = nout1 // _LANES

    src = edge_index[0].astype(jnp.int32)
    dst = edge_index[1].astype(jnp.int32)
    w123t = jnp.concatenate([fc1_1_w, fc1_2_w, fc1_3_w], axis=0).T
    w4t = fc1_4_w.T
    wz = conv_weight.transpose(1, 0, 2).reshape(ninp, d)
    s_idx = np.arange(d)
    blk, t = s_idx // 32, s_idx % 32
    perm = blk * 32 + (t % 2) * 16 + t // 2
    wz = wz[:, perm]

    be = 4000
    ea = pl.pallas_call(
        _edge_mlp_body,
        grid=(e // be,),
        in_specs=[
            pl.BlockSpec((be, edge_attr.shape[1]), lambda i: (i, 0)),
            pl.BlockSpec(w123t.shape, lambda i: (0, 0)),
            pl.BlockSpec(w4t.shape, lambda i: (0, 0)),
        ],
        out_specs=pl.BlockSpec((be, k_sup), lambda i: (i, 0)),
        out_shape=jax.ShapeDtypeStruct((e, k_sup), jnp.float32),
    )(edge_attr, w123t, w4t)

    bn = 2000
    z, r = pl.pallas_call(
        _project_body,
        grid=(n // bn,),
        in_specs=[
            pl.BlockSpec((bn, ninp), lambda i: (i, 0)),
            pl.BlockSpec((ninp, d), lambda i: (0, 0)),
            pl.BlockSpec((ninp, nout2), lambda i: (0, 0)),
            pl.BlockSpec((1, nout2), lambda i: (0, 0)),
            pl.BlockSpec((ninp, nout2), lambda i: (0, 0)),
            pl.BlockSpec((1, nout2), lambda i: (0, 0)),
        ],
        out_specs=[
            pl.BlockSpec((bn, d), lambda i: (i, 0)),
            pl.BlockSpec((bn, nout2), lambda i: (i, 0)),
        ],
        out_shape=[
            jax.ShapeDtypeStruct((n, d), jnp.bfloat16),
            jax.ShapeDtypeStruct((n, nout2), jnp.float32),
        ],
    )(x, wz, fc11_w.T, fc11_b.reshape(1, -1), fc12_w.T, fc12_b.reshape(1, -1))
    z = jax.lax.bitcast_convert_type(z.reshape(n, d // 2, 2), jnp.int32)

    chunk = 8
    sup = 40
    cps = sup // chunk
    e_per_w = e // _NW
    nsup = e_per_w // sup
    n_pad = ((n + 8 * _NS - 1) // (8 * _NS)) * (8 * _NS)
    rows_per_s = n_pad // _NS
    nzb = rows_per_s // sup

    mesh = plsc.VectorSubcoreMesh(core_axis_name="c", subcore_axis_name="s")

    @functools.partial(
        pl.kernel,
        out_type=jax.ShapeDtypeStruct((_NC, n_pad, nout1), jnp.float32),
        mesh=mesh,
        scratch_types=[
            pltpu.VMEM((sup,), jnp.int32),
            pltpu.VMEM((sup,), jnp.int32),
            pltpu.VMEM((sup, k_sup), jnp.float32),
            pltpu.VMEM((chunk, d // 2), jnp.int32),
            pltpu.VMEM((chunk, d // 2), jnp.int32),
            pltpu.VMEM((chunk, nout1), jnp.float32),
            pltpu.VMEM((chunk,), jnp.int32),
            pltpu.VMEM((chunk,), jnp.int32),
            pltpu.VMEM((chunk,), jnp.int32),
            pltpu.VMEM_SHARED((n_pad, nout1), jnp.float32),
            pltpu.SemaphoreType.DMA,
            pltpu.SemaphoreType.DMA,
        ],
        compiler_params=pltpu.CompilerParams(needs_layout_passes=False),
    )
    def _sc_spect(src_hbm, dst_hbm, ea_hbm, z_hbm, out_hbm,
                  src_v, dst_v, ea_v, z_a, z_b, y_v, idx_a, idx_b, dbuf,
                  acc_sh, sem_a, sem_b):
        cid = lax.axis_index("c")
        sid = lax.axis_index("s")
        wid = sid * _NC + cid
        zvec = jnp.zeros((_LANES,), jnp.float32)
        zbufs = (z_a, z_b)
        sems = (sem_a, sem_b)
        ibufs = (idx_a, idx_b)

        def _zero_row(rr, carry):
            for f in range(nf):
                y_v[rr, pl.ds(f * _LANES, _LANES)] = zvec
            return carry

        lax.fori_loop(0, chunk, _zero_row, 0)

        def _zero_cp(j, carry):
            pltpu.sync_copy(
                y_v, acc_sh.at[pl.ds(sid * rows_per_s + j * chunk, chunk)])
            return carry

        lax.fori_loop(0, rows_per_s // chunk, _zero_cp, 0)
        plsc.subcore_barrier()

        def _sup_body(sc, carry):
            base = pl.multiple_of(wid * e_per_w + sc * sup, 8)
            pltpu.sync_copy(ea_hbm.at[pl.ds(base, sup)], ea_v)

            descs = [None] * cps
            pltpu.sync_copy(src_hbm.at[pl.ds(base, chunk)], ibufs[0])
            descs[0] = pltpu.async_copy(
                z_hbm.at[ibufs[0]], zbufs[0], sems[0])
            for kc in range(cps):
                if kc + 1 < cps:
                    b1 = (kc + 1) % 2
                    pltpu.sync_copy(
                        src_hbm.at[pl.ds(base + (kc + 1) * chunk, chunk)],
                        ibufs[b1])
                    descs[kc + 1] = pltpu.async_copy(
                        z_hbm.at[ibufs[b1]], zbufs[b1], sems[b1])
                pltpu.sync_copy(
                    dst_hbm.at[pl.ds(base + kc * chunk, chunk)], dbuf)
                descs[kc].wait()
                zv = zbufs[kc % 2]

                def _edge(ee, ecarry, _kc=kc, _zv=zv):
                    ea_row = ea_v[_kc * chunk + ee, :]
                    accs = [zvec] * nf
                    for i in range(k_sup):
                        a = ea_row.at[jnp.full((_LANES,), i, jnp.int32)].get(
                            mode="promise_in_bounds")
                        for g in range(nf // 2):
                            w = _zv[ee, pl.ds((i * (nf // 2) + g) * _LANES,
                                              _LANES)]
                            lo, hi = plsc.unpack(
                                plsc.bitcast(w, jnp.bfloat16),
                                format=plsc.PackFormat.INTERLEAVED,
                                preferred_element_type=jnp.float32)
                            accs[2 * g] = accs[2 * g] + a * lo
                            accs[2 * g + 1] = accs[2 * g + 1] + a * hi
                    for f in range(nf):
                        y_v[ee, pl.ds(f * _LANES, _LANES)] = accs[f]
                    return ecarry

                lax.fori_loop(0, chunk, _edge, 0)
                pltpu.sync_copy(y_v, acc_sh.at[dbuf], add=True)
            return carry

        lax.fori_loop(0, nsup, _sup_body, 0)

        plsc.subcore_barrier()
        pltpu.sync_copy(acc_sh.at[pl.ds(sid * rows_per_s, rows_per_s)],
                        out_hbm.at[cid, pl.ds(sid * rows_per_s, rows_per_s)])

    partials = _sc_spect(src, dst, ea, z)[:, :n, :]

    out = pl.pallas_call(
        _combine_body,
        grid=(n // bn,),
        in_specs=[
            pl.BlockSpec((bn, nout1), lambda i: (i, 0)),
            pl.BlockSpec((bn, nout1), lambda i: (i, 0)),
            pl.BlockSpec((1, nout1), lambda i: (0, 0)),
            pl.BlockSpec((bn, nout2), lambda i: (i, 0)),
        ],
        out_specs=pl.BlockSpec((bn, nout1 + nout2), lambda i: (i, 0)),
        out_shape=jax.ShapeDtypeStruct((n, nout1 + nout2), jnp.float32),
    )(partials[0], partials[1], conv_bias.reshape(1, -1), r)
    return out


# speedup vs baseline: 1.2988x; 1.2988x over previous
import functools

import numpy as np

import jax
import jax.numpy as jnp
from jax import lax
from jax.experimental import pallas as pl
from jax.experimental.pallas import tpu as pltpu
from jax.experimental.pallas import tpu_sc as plsc

_NC, _NS, _LANES = 2, 16, 16
_NW = _NC * _NS


def _edge_mlp_body(attr_ref, w123t_ref, w4t_ref, ea_ref):
    t = jnp.dot(attr_ref[...], w123t_ref[...], preferred_element_type=jnp.float32)
    h = jax.nn.relu(t[:, :32])
    g = jnp.tanh(t[:, 32:64]) * jnp.tanh(t[:, 64:96])
    tmp = jnp.concatenate([h, g], axis=1)
    ea_ref[...] = jax.nn.relu(
        jnp.dot(tmp, w4t_ref[...], preferred_element_type=jnp.float32))


def _project_body(x_ref, wz_ref, w11t_ref, b11_ref, w12t_ref, b12_ref,
                  z_ref, r_ref):
    x = x_ref[...]
    z_ref[...] = jnp.dot(x, wz_ref[...], preferred_element_type=jnp.float32)
    r_ref[...] = (
        jnp.tanh(jnp.dot(x, w11t_ref[...], preferred_element_type=jnp.float32)
                 + b11_ref[...])
        * jnp.tanh(jnp.dot(x, w12t_ref[...], preferred_element_type=jnp.float32)
                   + b12_ref[...]))


def _combine_body(p0_ref, p1_ref, bias_ref, r_ref, out_ref):
    left = jax.nn.relu(p0_ref[...] + p1_ref[...] + bias_ref[...])
    out_ref[...] = jnp.concatenate([left, r_ref[...]], axis=1)


def kernel(x, edge_index, edge_attr, fc1_1_w, fc1_2_w, fc1_3_w, fc1_4_w,
           conv_weight, conv_bias, fc11_w, fc11_b, fc12_w, fc12_b):
    n, ninp = x.shape
    e = edge_attr.shape[0]
    k_sup, _, nout1 = conv_weight.shape
    nout2 = fc11_w.shape[0]
    d = k_sup * nout1
    nf = nout1 // _LANES

    src = edge_index[0].astype(jnp.int32)
    dst = edge_index[1].astype(jnp.int32)
    w123t = jnp.concatenate([fc1_1_w, fc1_2_w, fc1_3_w], axis=0).T
    w4t = fc1_4_w.T
    wz = conv_weight.transpose(1, 0, 2).reshape(ninp, d)

    chunk_c, sup_c = 8, 40

    be = 4000
    ea = pl.pallas_call(
        _edge_mlp_body,
        grid=(e // be,),
        in_specs=[
            pl.BlockSpec((be, edge_attr.shape[1]), lambda i: (i, 0)),
            pl.BlockSpec(w123t.shape, lambda i: (0, 0)),
            pl.BlockSpec(w4t.shape, lambda i: (0, 0)),
        ],
        out_specs=pl.BlockSpec((be, k_sup), lambda i: (i, 0)),
        out_shape=jax.ShapeDtypeStruct((e, k_sup), jnp.float32),
    )(edge_attr, w123t, w4t)

    bn = 2000
    z, r = pl.pallas_call(
        _project_body,
        grid=(n // bn,),
        in_specs=[
            pl.BlockSpec((bn, ninp), lambda i: (i, 0)),
            pl.BlockSpec((ninp, d), lambda i: (0, 0)),
            pl.BlockSpec((ninp, nout2), lambda i: (0, 0)),
            pl.BlockSpec((1, nout2), lambda i: (0, 0)),
            pl.BlockSpec((ninp, nout2), lambda i: (0, 0)),
            pl.BlockSpec((1, nout2), lambda i: (0, 0)),
        ],
        out_specs=[
            pl.BlockSpec((bn, d), lambda i: (i, 0)),
            pl.BlockSpec((bn, nout2), lambda i: (i, 0)),
        ],
        out_shape=[
            jax.ShapeDtypeStruct((n, d), jnp.float32),
            jax.ShapeDtypeStruct((n, nout2), jnp.float32),
        ],
    )(x, wz, fc11_w.T, fc11_b.reshape(1, -1), fc12_w.T, fc12_b.reshape(1, -1))
    src3 = src.reshape(e // sup_c, sup_c // chunk_c, chunk_c)
    dst3 = dst.reshape(e // sup_c, sup_c // chunk_c, chunk_c)

    chunk = 8
    sup = 40
    cps = sup // chunk
    e_per_w = e // _NW
    nsup = e_per_w // sup
    n_pad = ((n + 8 * _NS - 1) // (8 * _NS)) * (8 * _NS)
    rows_per_s = n_pad // _NS
    nzb = rows_per_s // sup

    mesh = plsc.VectorSubcoreMesh(core_axis_name="c", subcore_axis_name="s")

    @functools.partial(
        pl.kernel,
        out_type=jax.ShapeDtypeStruct((_NC, n_pad, nout1), jnp.float32),
        mesh=mesh,
        scratch_types=[
            pltpu.VMEM((cps, chunk), jnp.int32),
            pltpu.VMEM((cps, chunk), jnp.int32),
            pltpu.VMEM((sup, k_sup), jnp.float32),
            pltpu.VMEM((chunk, d), jnp.float32),
            pltpu.VMEM((chunk, d), jnp.float32),
            pltpu.VMEM((chunk, nout1), jnp.float32),
            pltpu.VMEM_SHARED((n_pad, nout1), jnp.float32),
            pltpu.SemaphoreType.DMA,
            pltpu.SemaphoreType.DMA,
        ],
    )
    def _sc_spect(src_hbm, dst_hbm, ea_hbm, z_hbm, out_hbm,
                  src_v, dst_v, ea_v, z_a, z_b, y_v,
                  acc_sh, sem_a, sem_b):
        cid = lax.axis_index("c")
        sid = lax.axis_index("s")
        wid = sid * _NC + cid
        zvec = jnp.zeros((_LANES,), jnp.float32)
        zbufs = (z_a, z_b)
        sems = (sem_a, sem_b)

        def _zero_row(rr, carry):
            for f in range(nf):
                y_v[rr, pl.ds(f * _LANES, _LANES)] = zvec
            return carry

        lax.fori_loop(0, chunk, _zero_row, 0)

        def _zero_cp(j, carry):
            pltpu.sync_copy(
                y_v, acc_sh.at[pl.ds(sid * rows_per_s + j * chunk, chunk)])
            return carry

        lax.fori_loop(0, rows_per_s // chunk, _zero_cp, 0)
        plsc.subcore_barrier()

        def _sup_body(sc, carry):
            base = pl.multiple_of(wid * e_per_w + sc * sup, 8)
            gsup = wid * nsup + sc
            pltpu.sync_copy(ea_hbm.at[pl.ds(base, sup)], ea_v)
            pltpu.sync_copy(src_hbm.at[gsup], src_v)
            pltpu.sync_copy(dst_hbm.at[gsup], dst_v)

            descs = [None] * cps
            descs[0] = pltpu.async_copy(
                z_hbm.at[src_v.at[0]], zbufs[0], sems[0])
            for kc in range(cps):
                if kc + 1 < cps:
                    b1 = (kc + 1) % 2
                    descs[kc + 1] = pltpu.async_copy(
                        z_hbm.at[src_v.at[kc + 1]], zbufs[b1], sems[b1])
                descs[kc].wait()
                zv = zbufs[kc % 2]

                def _edge(ee, ecarry, _kc=kc, _zv=zv):
                    ea_row = ea_v[_kc * chunk + ee, :]
                    accs = [zvec] * nf
                    for i in range(k_sup):
                        a = ea_row.at[jnp.full((_LANES,), i, jnp.int32)].get(
                            mode="promise_in_bounds")
                        for f in range(nf):
                            accs[f] = accs[f] + a * _zv[
                                ee, pl.ds(i * nout1 + f * _LANES, _LANES)]
                    for f in range(nf):
                        y_v[ee, pl.ds(f * _LANES, _LANES)] = accs[f]
                    return ecarry

                lax.fori_loop(0, chunk, _edge, 0)
                pltpu.sync_copy(y_v, acc_sh.at[dst_v.at[kc]], add=True)
            return carry

        lax.fori_loop(0, nsup, _sup_body, 0)

        plsc.subcore_barrier()
        pltpu.sync_copy(acc_sh.at[pl.ds(sid * rows_per_s, rows_per_s)],
                        out_hbm.at[cid, pl.ds(sid * rows_per_s, rows_per_s)])

    partials = _sc_spect(src3, dst3, ea, z)[:, :n, :]

    out = pl.pallas_call(
        _combine_body,
        grid=(n // bn,),
        in_specs=[
            pl.BlockSpec((bn, nout1), lambda i: (i, 0)),
            pl.BlockSpec((bn, nout1), lambda i: (i, 0)),
            pl.BlockSpec((1, nout1), lambda i: (0, 0)),
            pl.BlockSpec((bn, nout2), lambda i: (i, 0)),
        ],
        out_specs=pl.BlockSpec((bn, nout1 + nout2), lambda i: (i, 0)),
        out_shape=jax.ShapeDtypeStruct((n, nout1 + nout2), jnp.float32),
    )(partials[0], partials[1], conv_bias.reshape(1, -1), r)
    return out


# speedup vs baseline: 1.3438x; 1.0347x over previous
import functools

import numpy as np

import jax
import jax.numpy as jnp
from jax import lax
from jax.experimental import pallas as pl
from jax.experimental.pallas import tpu as pltpu
from jax.experimental.pallas import tpu_sc as plsc

_NC, _NS, _LANES = 2, 16, 16
_NW = _NC * _NS


def _edge_mlp_body(attr_ref, w123t_ref, w4t_ref, ea_ref):
    t = jnp.dot(attr_ref[...], w123t_ref[...], preferred_element_type=jnp.float32)
    h = jax.nn.relu(t[:, :32])
    g = jnp.tanh(t[:, 32:64]) * jnp.tanh(t[:, 64:96])
    tmp = jnp.concatenate([h, g], axis=1)
    ea_ref[...] = jax.nn.relu(
        jnp.dot(tmp, w4t_ref[...], preferred_element_type=jnp.float32))


def _project_body(x_ref, wz_ref, w11t_ref, b11_ref, w12t_ref, b12_ref,
                  z_ref, r_ref):
    x = x_ref[...]
    z_ref[...] = jnp.dot(x, wz_ref[...], preferred_element_type=jnp.float32)
    r_ref[...] = (
        jnp.tanh(jnp.dot(x, w11t_ref[...], preferred_element_type=jnp.float32)
                 + b11_ref[...])
        * jnp.tanh(jnp.dot(x, w12t_ref[...], preferred_element_type=jnp.float32)
                   + b12_ref[...]))


def _combine_body(p0_ref, p1_ref, bias_ref, r_ref, out_ref):
    left = jax.nn.relu(p0_ref[...] + p1_ref[...] + bias_ref[...])
    out_ref[...] = jnp.concatenate([left, r_ref[...]], axis=1)


def kernel(x, edge_index, edge_attr, fc1_1_w, fc1_2_w, fc1_3_w, fc1_4_w,
           conv_weight, conv_bias, fc11_w, fc11_b, fc12_w, fc12_b):
    n, ninp = x.shape
    e = edge_attr.shape[0]
    k_sup, _, nout1 = conv_weight.shape
    nout2 = fc11_w.shape[0]
    d = k_sup * nout1
    nf = nout1 // _LANES

    src = edge_index[0].astype(jnp.int32)
    dst = edge_index[1].astype(jnp.int32)
    w123t = jnp.concatenate([fc1_1_w, fc1_2_w, fc1_3_w], axis=0).T
    w4t = fc1_4_w.T
    wz = conv_weight.transpose(1, 0, 2).reshape(ninp, d)

    chunk_c, sup_c = 8, 40

    be = 4000
    ea = pl.pallas_call(
        _edge_mlp_body,
        grid=(e // be,),
        in_specs=[
            pl.BlockSpec((be, edge_attr.shape[1]), lambda i: (i, 0)),
            pl.BlockSpec(w123t.shape, lambda i: (0, 0)),
            pl.BlockSpec(w4t.shape, lambda i: (0, 0)),
        ],
        out_specs=pl.BlockSpec((be, k_sup), lambda i: (i, 0)),
        out_shape=jax.ShapeDtypeStruct((e, k_sup), jnp.float32),
    )(edge_attr, w123t, w4t)

    bn = 2000
    z, r = pl.pallas_call(
        _project_body,
        grid=(n // bn,),
        in_specs=[
            pl.BlockSpec((bn, ninp), lambda i: (i, 0)),
            pl.BlockSpec((ninp, d), lambda i: (0, 0)),
            pl.BlockSpec((ninp, nout2), lambda i: (0, 0)),
            pl.BlockSpec((1, nout2), lambda i: (0, 0)),
            pl.BlockSpec((ninp, nout2), lambda i: (0, 0)),
            pl.BlockSpec((1, nout2), lambda i: (0, 0)),
        ],
        out_specs=[
            pl.BlockSpec((bn, d), lambda i: (i, 0)),
            pl.BlockSpec((bn, nout2), lambda i: (i, 0)),
        ],
        out_shape=[
            jax.ShapeDtypeStruct((n, d), jnp.float32),
            jax.ShapeDtypeStruct((n, nout2), jnp.float32),
        ],
    )(x, wz, fc11_w.T, fc11_b.reshape(1, -1), fc12_w.T, fc12_b.reshape(1, -1))
    src3 = src.reshape(e // sup_c, sup_c // chunk_c, chunk_c)
    dst3 = dst.reshape(e // sup_c, sup_c // chunk_c, chunk_c)

    chunk = 8
    sup = 40
    cps = sup // chunk
    e_per_w = e // _NW
    nsup = e_per_w // sup
    n_pad = ((n + 8 * _NS - 1) // (8 * _NS)) * (8 * _NS)
    rows_per_s = n_pad // _NS
    nzb = rows_per_s // sup

    mesh = plsc.VectorSubcoreMesh(core_axis_name="c", subcore_axis_name="s")

    @functools.partial(
        pl.kernel,
        out_type=jax.ShapeDtypeStruct((_NC, n_pad, nout1), jnp.float32),
        mesh=mesh,
        scratch_types=[
            pltpu.VMEM((cps, chunk), jnp.int32),
            pltpu.VMEM((cps, chunk), jnp.int32),
            pltpu.VMEM((sup, k_sup), jnp.float32),
            pltpu.VMEM((chunk, d), jnp.float32),
            pltpu.VMEM((chunk, d), jnp.float32),
            pltpu.VMEM((chunk, nout1), jnp.float32),
            pltpu.VMEM_SHARED((n_pad, nout1), jnp.float32),
            pltpu.SemaphoreType.DMA,
            pltpu.SemaphoreType.DMA,
            pltpu.SemaphoreType.DMA,
        ],
    )
    def _sc_spect(src_hbm, dst_hbm, ea_hbm, z_hbm, out_hbm,
                  src_v, dst_v, ea_v, z_a, z_b, y_v,
                  acc_sh, sem_a, sem_b, sem_s):
        cid = lax.axis_index("c")
        sid = lax.axis_index("s")
        wid = sid * _NC + cid
        zvec = jnp.zeros((_LANES,), jnp.float32)
        zbufs = (z_a, z_b)
        sems = (sem_a, sem_b)

        def _zero_row(rr, carry):
            for f in range(nf):
                y_v[rr, pl.ds(f * _LANES, _LANES)] = zvec
            return carry

        lax.fori_loop(0, chunk, _zero_row, 0)

        def _zero_cp(j, carry):
            pltpu.sync_copy(
                y_v, acc_sh.at[pl.ds(sid * rows_per_s + j * chunk, chunk)])
            return carry

        lax.fori_loop(0, rows_per_s // chunk, _zero_cp, 0)
        plsc.subcore_barrier()

        def _sup_body(sc, carry):
            base = pl.multiple_of(wid * e_per_w + sc * sup, 8)
            gsup = wid * nsup + sc
            st1 = pltpu.async_copy(ea_hbm.at[pl.ds(base, sup)], ea_v, sem_s)
            st2 = pltpu.async_copy(src_hbm.at[gsup], src_v, sem_s)
            st3 = pltpu.async_copy(dst_hbm.at[gsup], dst_v, sem_s)
            st1.wait()
            st2.wait()
            st3.wait()

            descs = [None] * cps
            descs[0] = pltpu.async_copy(
                z_hbm.at[src_v.at[0]], zbufs[0], sems[0])
            for kc in range(cps):
                if kc + 1 < cps:
                    b1 = (kc + 1) % 2
                    descs[kc + 1] = pltpu.async_copy(
                        z_hbm.at[src_v.at[kc + 1]], zbufs[b1], sems[b1])
                descs[kc].wait()
                zv = zbufs[kc % 2]

                def _edge2(e2, ecarry, _kc=kc, _zv=zv):
                    for u in range(2):
                        ee = 2 * e2 + u
                        ea_row = ea_v[_kc * chunk + ee, :]
                        accs = [zvec] * nf
                        for i in range(k_sup):
                            a = ea_row.at[
                                jnp.full((_LANES,), i, jnp.int32)].get(
                                    mode="promise_in_bounds")
                            for f in range(nf):
                                accs[f] = accs[f] + a * _zv[
                                    ee, pl.ds(i * nout1 + f * _LANES, _LANES)]
                        for f in range(nf):
                            y_v[ee, pl.ds(f * _LANES, _LANES)] = accs[f]
                    return ecarry

                lax.fori_loop(0, chunk // 2, _edge2, 0)
                pltpu.sync_copy(y_v, acc_sh.at[dst_v.at[kc]], add=True)
            return carry

        lax.fori_loop(0, nsup, _sup_body, 0)

        plsc.subcore_barrier()
        pltpu.sync_copy(acc_sh.at[pl.ds(sid * rows_per_s, rows_per_s)],
                        out_hbm.at[cid, pl.ds(sid * rows_per_s, rows_per_s)])

    partials = _sc_spect(src3, dst3, ea, z)[:, :n, :]

    out = pl.pallas_call(
        _combine_body,
        grid=(n // bn,),
        in_specs=[
            pl.BlockSpec((bn, nout1), lambda i: (i, 0)),
            pl.BlockSpec((bn, nout1), lambda i: (i, 0)),
            pl.BlockSpec((1, nout1), lambda i: (0, 0)),
            pl.BlockSpec((bn, nout2), lambda i: (i, 0)),
        ],
        out_specs=pl.BlockSpec((bn, nout1 + nout2), lambda i: (i, 0)),
        out_shape=jax.ShapeDtypeStruct((n, nout1 + nout2), jnp.float32),
    )(partials[0], partials[1], conv_bias.reshape(1, -1), r)
    return out


# speedup vs baseline: 1.3673x; 1.0175x over previous
import functools

import numpy as np

import jax
import jax.numpy as jnp
from jax import lax
from jax.experimental import pallas as pl
from jax.experimental.pallas import tpu as pltpu
from jax.experimental.pallas import tpu_sc as plsc

_NC, _NS, _LANES = 2, 16, 16
_NW = _NC * _NS


def _edge_mlp_body(attr_ref, w123t_ref, w4t_ref, ea_ref):
    t = jnp.dot(attr_ref[...], w123t_ref[...], preferred_element_type=jnp.float32)
    h = jax.nn.relu(t[:, :32])
    g = jnp.tanh(t[:, 32:64]) * jnp.tanh(t[:, 64:96])
    tmp = jnp.concatenate([h, g], axis=1)
    ea_ref[...] = jax.nn.relu(
        jnp.dot(tmp, w4t_ref[...], preferred_element_type=jnp.float32))


def _project_body(x_ref, wz_ref, w11t_ref, b11_ref, w12t_ref, b12_ref,
                  z_ref, r_ref):
    x = x_ref[...]
    z_ref[...] = jnp.dot(x, wz_ref[...], preferred_element_type=jnp.float32)
    r_ref[...] = (
        jnp.tanh(jnp.dot(x, w11t_ref[...], preferred_element_type=jnp.float32)
                 + b11_ref[...])
        * jnp.tanh(jnp.dot(x, w12t_ref[...], preferred_element_type=jnp.float32)
                   + b12_ref[...]))


def _combine_body(p0_ref, p1_ref, bias_ref, r_ref, out_ref):
    left = jax.nn.relu(p0_ref[...] + p1_ref[...] + bias_ref[...])
    out_ref[...] = jnp.concatenate([left, r_ref[...]], axis=1)


def kernel(x, edge_index, edge_attr, fc1_1_w, fc1_2_w, fc1_3_w, fc1_4_w,
           conv_weight, conv_bias, fc11_w, fc11_b, fc12_w, fc12_b):
    n, ninp = x.shape
    e = edge_attr.shape[0]
    k_sup, _, nout1 = conv_weight.shape
    nout2 = fc11_w.shape[0]
    d = k_sup * nout1
    nf = nout1 // _LANES

    src = edge_index[0].astype(jnp.int32)
    dst = edge_index[1].astype(jnp.int32)
    w123t = jnp.concatenate([fc1_1_w, fc1_2_w, fc1_3_w], axis=0).T
    w4t = fc1_4_w.T
    wz = conv_weight.transpose(1, 0, 2).reshape(ninp, d)

    chunk_c, sup_c = 8, 40

    be = 4000
    ea = pl.pallas_call(
        _edge_mlp_body,
        grid=(e // be,),
        in_specs=[
            pl.BlockSpec((be, edge_attr.shape[1]), lambda i: (i, 0)),
            pl.BlockSpec(w123t.shape, lambda i: (0, 0)),
            pl.BlockSpec(w4t.shape, lambda i: (0, 0)),
        ],
        out_specs=pl.BlockSpec((be, k_sup), lambda i: (i, 0)),
        out_shape=jax.ShapeDtypeStruct((e, k_sup), jnp.float32),
    )(edge_attr, w123t, w4t)

    bn = 2000
    z, r = pl.pallas_call(
        _project_body,
        grid=(n // bn,),
        in_specs=[
            pl.BlockSpec((bn, ninp), lambda i: (i, 0)),
            pl.BlockSpec((ninp, d), lambda i: (0, 0)),
            pl.BlockSpec((ninp, nout2), lambda i: (0, 0)),
            pl.BlockSpec((1, nout2), lambda i: (0, 0)),
            pl.BlockSpec((ninp, nout2), lambda i: (0, 0)),
            pl.BlockSpec((1, nout2), lambda i: (0, 0)),
        ],
        out_specs=[
            pl.BlockSpec((bn, d), lambda i: (i, 0)),
            pl.BlockSpec((bn, nout2), lambda i: (i, 0)),
        ],
        out_shape=[
            jax.ShapeDtypeStruct((n, d), jnp.float32),
            jax.ShapeDtypeStruct((n, nout2), jnp.float32),
        ],
    )(x, wz, fc11_w.T, fc11_b.reshape(1, -1), fc12_w.T, fc12_b.reshape(1, -1))
    src3 = src.reshape(e // sup_c, sup_c // chunk_c, chunk_c)
    dst3 = dst.reshape(e // sup_c, sup_c // chunk_c, chunk_c)

    chunk = 8
    sup = 40
    cps = sup // chunk
    e_per_w = e // _NW
    nsup = e_per_w // sup
    n_pad = ((n + 8 * _NS - 1) // (8 * _NS)) * (8 * _NS)
    rows_per_s = n_pad // _NS
    nzb = rows_per_s // sup

    mesh = plsc.VectorSubcoreMesh(core_axis_name="c", subcore_axis_name="s")

    @functools.partial(
        pl.kernel,
        out_type=jax.ShapeDtypeStruct((_NC, n_pad, nout1), jnp.float32),
        mesh=mesh,
        scratch_types=[
            pltpu.VMEM((cps, chunk), jnp.int32),
            pltpu.VMEM((cps, chunk), jnp.int32),
            pltpu.VMEM((sup, k_sup), jnp.float32),
            pltpu.VMEM((chunk, d), jnp.float32),
            pltpu.VMEM((chunk, d), jnp.float32),
            pltpu.VMEM((chunk, nout1), jnp.float32),
            pltpu.VMEM((chunk, nout1), jnp.float32),
            pltpu.VMEM_SHARED((n_pad, nout1), jnp.float32),
            pltpu.SemaphoreType.DMA,
            pltpu.SemaphoreType.DMA,
            pltpu.SemaphoreType.DMA,
            pltpu.SemaphoreType.DMA,
        ],
    )
    def _sc_spect(src_hbm, dst_hbm, ea_hbm, z_hbm, out_hbm,
                  src_v, dst_v, ea_v, z_a, z_b, y_v, y_w,
                  acc_sh, sem_a, sem_b, sem_s, sem_y):
        cid = lax.axis_index("c")
        sid = lax.axis_index("s")
        wid = sid * _NC + cid
        zvec = jnp.zeros((_LANES,), jnp.float32)
        zbufs = (z_a, z_b)
        sems = (sem_a, sem_b)
        ybufs = (y_v, y_w)

        def _zero_row(rr, carry):
            for f in range(nf):
                y_v[rr, pl.ds(f * _LANES, _LANES)] = zvec
            return carry

        lax.fori_loop(0, chunk, _zero_row, 0)

        def _zero_cp(j, carry):
            pltpu.sync_copy(
                y_v, acc_sh.at[pl.ds(sid * rows_per_s + j * chunk, chunk)])
            return carry

        lax.fori_loop(0, rows_per_s // chunk, _zero_cp, 0)
        plsc.subcore_barrier()

        def _sup_body(sc, carry):
            base = pl.multiple_of(wid * e_per_w + sc * sup, 8)
            gsup = wid * nsup + sc
            st1 = pltpu.async_copy(ea_hbm.at[pl.ds(base, sup)], ea_v, sem_s)
            st2 = pltpu.async_copy(src_hbm.at[gsup], src_v, sem_s)
            st3 = pltpu.async_copy(dst_hbm.at[gsup], dst_v, sem_s)
            st1.wait()
            st2.wait()
            st3.wait()

            descs = [None] * cps
            sdescs = [None] * cps
            descs[0] = pltpu.async_copy(
                z_hbm.at[src_v.at[0]], zbufs[0], sems[0])
            for kc in range(cps):
                if kc + 1 < cps:
                    b1 = (kc + 1) % 2
                    descs[kc + 1] = pltpu.async_copy(
                        z_hbm.at[src_v.at[kc + 1]], zbufs[b1], sems[b1])
                descs[kc].wait()
                zv = zbufs[kc % 2]

                yv = ybufs[kc % 2]

                def _edge2(e2, ecarry, _kc=kc, _zv=zv, _yv=yv):
                    for u in range(2):
                        ee = 2 * e2 + u
                        ea_row = ea_v[_kc * chunk + ee, :]
                        accs = [zvec] * nf
                        for i in range(k_sup):
                            a = ea_row.at[
                                jnp.full((_LANES,), i, jnp.int32)].get(
                                    mode="promise_in_bounds")
                            for f in range(nf):
                                accs[f] = accs[f] + a * _zv[
                                    ee, pl.ds(i * nout1 + f * _LANES, _LANES)]
                        for f in range(nf):
                            _yv[ee, pl.ds(f * _LANES, _LANES)] = accs[f]
                    return ecarry

                lax.fori_loop(0, chunk // 2, _edge2, 0)
                if kc >= 1:
                    sdescs[kc - 1].wait()
                sdescs[kc] = pltpu.async_copy(
                    yv, acc_sh.at[dst_v.at[kc]], sem_y, add=True)
            sdescs[cps - 1].wait()
            return carry

        lax.fori_loop(0, nsup, _sup_body, 0)

        plsc.subcore_barrier()
        pltpu.sync_copy(acc_sh.at[pl.ds(sid * rows_per_s, rows_per_s)],
                        out_hbm.at[cid, pl.ds(sid * rows_per_s, rows_per_s)])

    partials = _sc_spect(src3, dst3, ea, z)[:, :n, :]

    out = pl.pallas_call(
        _combine_body,
        grid=(n // bn,),
        in_specs=[
            pl.BlockSpec((bn, nout1), lambda i: (i, 0)),
            pl.BlockSpec((bn, nout1), lambda i: (i, 0)),
            pl.BlockSpec((1, nout1), lambda i: (0, 0)),
            pl.BlockSpec((bn, nout2), lambda i: (i, 0)),
        ],
        out_specs=pl.BlockSpec((bn, nout1 + nout2), lambda i: (i, 0)),
        out_shape=jax.ShapeDtypeStruct((n, nout1 + nout2), jnp.float32),
    )(partials[0], partials[1], conv_bias.reshape(1, -1), r)
    return out
